# Initial kernel scaffold; baseline (speedup 1.0000x reference)
import functools

import jax
import jax.numpy as jnp
from jax import lax
from jax.experimental import pallas as pl
from jax.experimental.pallas import tpu as pltpu
from jax.experimental.pallas import tpu_sc as plsc

N_ATOMS = 10000
N_PAIRS = 160000
NF = 128
N_DIST = 10
HARD_CUTOFF = 5.5
GN_EPS = 1e-05

C = 2000
NB = N_PAIRS // C


def _sc_gather(table, idx):
    info = plsc.get_sparse_core_info()
    nc, ns = info.num_cores, info.num_subcores
    nw = nc * ns
    b_per_w = N_PAIRS // nw
    ch = 200
    n_ch = b_per_w // ch
    mesh = plsc.VectorSubcoreMesh(core_axis_name="c", subcore_axis_name="s")

    @functools.partial(
        pl.kernel,
        mesh=mesh,
        out_type=jax.ShapeDtypeStruct((N_PAIRS, NF), jnp.float32),
        scratch_types=[
            pltpu.VMEM((ch,), jnp.int32),
            pltpu.VMEM((ch, NF), jnp.float32),
            pltpu.SemaphoreType.DMA,
        ],
    )
    def gather_kernel(table_hbm, idx_hbm, out_hbm, idx_v, rows_v, sem):
        wid = lax.axis_index("s") * nc + lax.axis_index("c")
        base = wid * b_per_w

        def body(j, carry):
            off = base + j * ch
            pltpu.sync_copy(idx_hbm.at[pl.ds(off, ch)], idx_v)
            pltpu.async_copy(table_hbm.at[idx_v], rows_v, sem).wait()
            pltpu.sync_copy(rows_v, out_hbm.at[pl.ds(off, ch)])
            return carry

        lax.fori_loop(0, n_ch, body, 0)

    return gather_kernel(table, idx)


def _tc_main_body(g_ref, d_ref, rh_ref, mu_ref, sg_ref, w_ref, pf_ref, out_ref, c_ref):
    step = pl.program_id(0)

    @pl.when(step == 0)
    def _():
        out_ref[...] = jnp.zeros_like(out_ref)
        c_ref[...] = jnp.zeros_like(c_ref)

    gc = g_ref[...]
    d = jnp.maximum(d_ref[0], 1e-6)
    dc = d.reshape(C, 1)
    invc = 1.0 / dc
    z = (invc - mu_ref[...]) / sg_ref[...]
    base = jnp.exp(-0.5 * z * z)
    cut = jnp.where(dc < HARD_CUTOFF,
                    0.5 * (jnp.cos(jnp.pi / HARD_CUTOFF * dc) + 1.0), 0.0)
    sense = base * (cut * cut)

    q = jnp.zeros((C, NF), jnp.float32)
    for s in range(N_DIST):
        ws = w_ref[s]
        gs = lax.dot_general(gc, ws, (((1,), (1,)), ((), ())),
                             preferred_element_type=jnp.float32)
        q = q + sense[:, s:s + 1] * gs

    rh = rh_ref[0]
    for t in range(4):
        c_ref[:, t, :] = rh[:, t:t + 1] * q

    def scatter(i, carry):
        a = pf_ref[0, 0, i]
        out_ref[a] = out_ref[a] + c_ref[i]
        return carry

    lax.fori_loop(0, C, scatter, 0)


def _tc_main(g, dist3, rhat3, pf3, mu2, sg2, int_w):
    return pl.pallas_call(
        _tc_main_body,
        grid=(NB,),
        in_specs=[
            pl.BlockSpec((C, NF), lambda i: (i, 0)),
            pl.BlockSpec((1, 1, C), lambda i: (i, 0, 0)),
            pl.BlockSpec((1, C, 8), lambda i: (i, 0, 0)),
            pl.BlockSpec((1, N_DIST), lambda i: (0, 0)),
            pl.BlockSpec((1, N_DIST), lambda i: (0, 0)),
            pl.BlockSpec((N_DIST, NF, NF), lambda i: (0, 0, 0)),
            pl.BlockSpec((1, 1, C), lambda i: (i, 0, 0), memory_space=pltpu.SMEM),
        ],
        out_specs=pl.BlockSpec((N_ATOMS, 8, NF), lambda i: (0, 0, 0)),
        out_shape=jax.ShapeDtypeStruct((N_ATOMS, 8, NF), jnp.float32),
        scratch_shapes=[pltpu.VMEM((C, 8, NF), jnp.float32)],
        compiler_params=pltpu.CompilerParams(
            dimension_semantics=("arbitrary",),
        ),
    )(g, dist3, rhat3, mu2, sg2, int_w, pf3)


AB = 1000


def _tc_tail_body(tf_ref, feat_ref, sw_ref, sb_ref, mw_ref, gg_ref, gb_ref, o_ref):
    inv1 = tf_ref[:, 0, :]
    inv2 = (tf_ref[:, 1, :] ** 2 + tf_ref[:, 2, :] ** 2 + tf_ref[:, 3, :] ** 2)
    acc = jnp.zeros((AB, NF), jnp.float32)
    for gidx, xg in ((0, inv1), (1, inv2)):
        m = jnp.mean(xg, axis=1, keepdims=True)
        xc = xg - m
        v = jnp.mean(xc * xc, axis=1, keepdims=True)
        xn = xc * lax.rsqrt(v + GN_EPS)
        xn = xn * gg_ref[gidx:gidx + 1, :] + gb_ref[gidx:gidx + 1, :]
        mg = mw_ref[:, gidx, :]
        acc = acc + jnp.dot(xn, mg, preferred_element_type=jnp.float32)
    selfp = lax.dot_general(feat_ref[...], sw_ref[...], (((1,), (1,)), ((), ())),
                            preferred_element_type=jnp.float32) + sb_ref[...]
    o_ref[...] = acc + selfp


def _tc_tail(tf, feat, sw, sb2, mw, gg2, gb2):
    nblk = N_ATOMS // AB
    return pl.pallas_call(
        _tc_tail_body,
        grid=(nblk,),
        in_specs=[
            pl.BlockSpec((AB, 8, NF), lambda i: (i, 0, 0)),
            pl.BlockSpec((AB, NF), lambda i: (i, 0)),
            pl.BlockSpec((NF, NF), lambda i: (0, 0)),
            pl.BlockSpec((1, NF), lambda i: (0, 0)),
            pl.BlockSpec((NF, 2, NF), lambda i: (0, 0, 0)),
            pl.BlockSpec((2, NF), lambda i: (0, 0)),
            pl.BlockSpec((2, NF), lambda i: (0, 0)),
        ],
        out_specs=pl.BlockSpec((AB, NF), lambda i: (i, 0)),
        out_shape=jax.ShapeDtypeStruct((N_ATOMS, NF), jnp.float32),
        compiler_params=pltpu.CompilerParams(
            dimension_semantics=("arbitrary",),
        ),
    )(tf, feat, sw, sb2, mw, gg2, gb2)


def kernel(in_features, pair_first, pair_second, dist_pairs, tensor_rhats,
           sense_mu, sense_sigma, int_weights, selfint_W, selfint_b,
           mixing_weights, gn_gamma, gn_beta):
    g = _sc_gather(in_features, pair_second)

    dist3 = dist_pairs.reshape(NB, 1, C)
    rhat8 = jnp.concatenate(
        [tensor_rhats, jnp.zeros((N_PAIRS, 4), jnp.float32)], axis=1)
    rhat3 = rhat8.reshape(NB, C, 8)
    pf3 = pair_first.reshape(NB, 1, C)
    mu2 = sense_mu.reshape(1, N_DIST)
    sg2 = sense_sigma.reshape(1, N_DIST)

    tf = _tc_main(g, dist3, rhat3, pf3, mu2, sg2, int_weights)

    out = _tc_tail(tf, in_features, selfint_W, selfint_b.reshape(1, NF),
                   mixing_weights, gn_gamma.reshape(2, NF),
                   gn_beta.reshape(2, NF))
    return out


# speedup vs baseline: 8.1060x; 8.1060x over previous
import functools

import jax
import jax.numpy as jnp
from jax import lax
from jax.experimental import pallas as pl
from jax.experimental.pallas import tpu as pltpu
from jax.experimental.pallas import tpu_sc as plsc

N_ATOMS = 10000
N_PAIRS = 160000
NF = 128
N_DIST = 10
HARD_CUTOFF = 5.5
GN_EPS = 1e-05

C = 1000
NB = N_PAIRS // C


def _sc_gather(table, idx):
    info = plsc.get_sparse_core_info()
    nc, ns = info.num_cores, info.num_subcores
    nw = nc * ns
    b_per_w = N_PAIRS // nw
    ch = 200
    n_ch = b_per_w // ch
    mesh = plsc.VectorSubcoreMesh(core_axis_name="c", subcore_axis_name="s")

    @functools.partial(
        pl.kernel,
        mesh=mesh,
        out_type=jax.ShapeDtypeStruct((N_PAIRS, NF), jnp.float32),
        scratch_types=[
            pltpu.VMEM((ch,), jnp.int32),
            pltpu.VMEM((ch, NF), jnp.float32),
            pltpu.SemaphoreType.DMA,
        ],
    )
    def gather_kernel(table_hbm, idx_hbm, out_hbm, idx_v, rows_v, sem):
        wid = lax.axis_index("s") * nc + lax.axis_index("c")
        base = wid * b_per_w

        def body(j, carry):
            off = base + j * ch
            pltpu.sync_copy(idx_hbm.at[pl.ds(off, ch)], idx_v)
            pltpu.async_copy(table_hbm.at[idx_v], rows_v, sem).wait()
            pltpu.sync_copy(rows_v, out_hbm.at[pl.ds(off, ch)])
            return carry

        lax.fori_loop(0, n_ch, body, 0)

    return gather_kernel(table, idx)


def _tc_main_body(g_ref, d_ref, rh_ref, mu_ref, sg_ref, w_ref, pf_ref, out_ref, c_ref):
    step = pl.program_id(0)

    @pl.when(step == 0)
    def _():
        out_ref[...] = jnp.zeros_like(out_ref)
        c_ref[...] = jnp.zeros_like(c_ref)

    gc = g_ref[...]
    d = jnp.maximum(d_ref[0], 1e-6)
    dc = d.reshape(C, 1)
    invc = 1.0 / dc
    z = (invc - mu_ref[...]) / sg_ref[...]
    base = jnp.exp(-0.5 * z * z)
    cut = jnp.where(dc < HARD_CUTOFF,
                    0.5 * (jnp.cos(jnp.pi / HARD_CUTOFF * dc) + 1.0), 0.0)
    sense = base * (cut * cut)

    q = jnp.zeros((C, NF), jnp.float32)
    for s in range(N_DIST):
        ws = w_ref[s]
        gs = lax.dot_general(gc, ws, (((1,), (1,)), ((), ())),
                             preferred_element_type=jnp.float32)
        q = q + sense[:, s:s + 1] * gs

    rh = rh_ref[0]
    for t in range(4):
        c_ref[:, t, :] = rh[:, t:t + 1] * q

    def scatter(i, carry):
        a = pf_ref[0, 0, i]
        out_ref[a] = out_ref[a] + c_ref[i]
        return carry

    lax.fori_loop(0, C, scatter, 0)


def _tc_main(g, dist3, rhat3, pf3, mu2, sg2, int_w):
    return pl.pallas_call(
        _tc_main_body,
        grid=(NB,),
        in_specs=[
            pl.BlockSpec((C, NF), lambda i: (i, 0)),
            pl.BlockSpec((1, 1, C), lambda i: (i, 0, 0)),
            pl.BlockSpec((1, C, 8), lambda i: (i, 0, 0)),
            pl.BlockSpec((1, N_DIST), lambda i: (0, 0)),
            pl.BlockSpec((1, N_DIST), lambda i: (0, 0)),
            pl.BlockSpec((N_DIST, NF, NF), lambda i: (0, 0, 0)),
            pl.BlockSpec((1, 1, C), lambda i: (i, 0, 0), memory_space=pltpu.SMEM),
        ],
        out_specs=pl.BlockSpec((N_ATOMS, 8, NF), lambda i: (0, 0, 0)),
        out_shape=jax.ShapeDtypeStruct((N_ATOMS, 8, NF), jnp.float32),
        scratch_shapes=[pltpu.VMEM((C, 8, NF), jnp.float32)],
        compiler_params=pltpu.CompilerParams(
            dimension_semantics=("arbitrary",),
        ),
    )(g, dist3, rhat3, mu2, sg2, int_w, pf3)


AB = 1000


def _tc_tail_body(tf_ref, feat_ref, sw_ref, sb_ref, mw_ref, gg_ref, gb_ref, o_ref):
    inv1 = tf_ref[:, 0, :]
    inv2 = (tf_ref[:, 1, :] ** 2 + tf_ref[:, 2, :] ** 2 + tf_ref[:, 3, :] ** 2)
    acc = jnp.zeros((AB, NF), jnp.float32)
    for gidx, xg in ((0, inv1), (1, inv2)):
        m = jnp.mean(xg, axis=1, keepdims=True)
        xc = xg - m
        v = jnp.mean(xc * xc, axis=1, keepdims=True)
        xn = xc * lax.rsqrt(v + GN_EPS)
        xn = xn * gg_ref[gidx:gidx + 1, :] + gb_ref[gidx:gidx + 1, :]
        mg = mw_ref[:, gidx, :]
        acc = acc + jnp.dot(xn, mg, preferred_element_type=jnp.float32)
    selfp = lax.dot_general(feat_ref[...], sw_ref[...], (((1,), (1,)), ((), ())),
                            preferred_element_type=jnp.float32) + sb_ref[...]
    o_ref[...] = acc + selfp


def _tc_tail(tf, feat, sw, sb2, mw, gg2, gb2):
    nblk = N_ATOMS // AB
    return pl.pallas_call(
        _tc_tail_body,
        grid=(nblk,),
        in_specs=[
            pl.BlockSpec((AB, 8, NF), lambda i: (i, 0, 0)),
            pl.BlockSpec((AB, NF), lambda i: (i, 0)),
            pl.BlockSpec((NF, NF), lambda i: (0, 0)),
            pl.BlockSpec((1, NF), lambda i: (0, 0)),
            pl.BlockSpec((NF, 2, NF), lambda i: (0, 0, 0)),
            pl.BlockSpec((2, NF), lambda i: (0, 0)),
            pl.BlockSpec((2, NF), lambda i: (0, 0)),
        ],
        out_specs=pl.BlockSpec((AB, NF), lambda i: (i, 0)),
        out_shape=jax.ShapeDtypeStruct((N_ATOMS, NF), jnp.float32),
        compiler_params=pltpu.CompilerParams(
            dimension_semantics=("arbitrary",),
        ),
    )(tf, feat, sw, sb2, mw, gg2, gb2)


def kernel(in_features, pair_first, pair_second, dist_pairs, tensor_rhats,
           sense_mu, sense_sigma, int_weights, selfint_W, selfint_b,
           mixing_weights, gn_gamma, gn_beta):
    g = _sc_gather(in_features, pair_second)

    dist3 = dist_pairs.reshape(NB, 1, C)
    rhat8 = jnp.concatenate(
        [tensor_rhats, jnp.zeros((N_PAIRS, 4), jnp.float32)], axis=1)
    rhat3 = rhat8.reshape(NB, C, 8)
    pf3 = pair_first.reshape(NB, 1, C)
    mu2 = sense_mu.reshape(1, N_DIST)
    sg2 = sense_sigma.reshape(1, N_DIST)

    tf = _tc_main(g, dist3, rhat3, pf3, mu2, sg2, int_weights)

    out = _tc_tail(tf, in_features, selfint_W, selfint_b.reshape(1, NF),
                   mixing_weights, gn_gamma.reshape(2, NF),
                   gn_beta.reshape(2, NF))
    return out


# speedup vs baseline: 11.5618x; 1.4263x over previous
import functools

import jax
import jax.numpy as jnp
from jax import lax
from jax.experimental import pallas as pl
from jax.experimental.pallas import tpu as pltpu
from jax.experimental.pallas import tpu_sc as plsc

N_ATOMS = 10000
N_PAIRS = 160000
NF = 128
N_DIST = 10
HARD_CUTOFF = 5.5
GN_EPS = 1e-05

C = 1000
NB = N_PAIRS // C


def _sc_gather(table, idx):
    info = plsc.get_sparse_core_info()
    nc, ns = info.num_cores, info.num_subcores
    nw = nc * ns
    b_per_w = N_PAIRS // nw
    ch = 200
    n_ch = b_per_w // ch
    mesh = plsc.VectorSubcoreMesh(core_axis_name="c", subcore_axis_name="s")

    @functools.partial(
        pl.kernel,
        mesh=mesh,
        out_type=jax.ShapeDtypeStruct((N_PAIRS, NF), jnp.float32),
        scratch_types=[
            pltpu.VMEM((ch,), jnp.int32),
            pltpu.VMEM((ch, NF), jnp.float32),
            pltpu.SemaphoreType.DMA,
        ],
    )
    def gather_kernel(table_hbm, idx_hbm, out_hbm, idx_v, rows_v, sem):
        wid = lax.axis_index("s") * nc + lax.axis_index("c")
        base = wid * b_per_w

        def body(j, carry):
            off = base + j * ch
            pltpu.sync_copy(idx_hbm.at[pl.ds(off, ch)], idx_v)
            pltpu.async_copy(table_hbm.at[idx_v], rows_v, sem).wait()
            pltpu.sync_copy(rows_v, out_hbm.at[pl.ds(off, ch)])
            return carry

        lax.fori_loop(0, n_ch, body, 0)

    return gather_kernel(table, idx)


def _tc_main_body(g_ref, d_ref, rh_ref, mu_ref, sg_ref, w_ref, pf_ref, out_ref, c_ref):
    step = pl.program_id(0)

    @pl.when(step == 0)
    def _():
        out_ref[...] = jnp.zeros_like(out_ref)
        c_ref[...] = jnp.zeros_like(c_ref)

    gc = g_ref[...]
    d = jnp.maximum(d_ref[0], 1e-6)
    dc = d.reshape(C, 1)
    invc = 1.0 / dc
    z = (invc - mu_ref[...]) / sg_ref[...]
    base = jnp.exp(-0.5 * z * z)
    cut = jnp.where(dc < HARD_CUTOFF,
                    0.5 * (jnp.cos(jnp.pi / HARD_CUTOFF * dc) + 1.0), 0.0)
    sense = base * (cut * cut)

    q = jnp.zeros((C, NF), jnp.float32)
    for s in range(N_DIST):
        ws = w_ref[s]
        gs = lax.dot_general(gc, ws, (((1,), (1,)), ((), ())),
                             preferred_element_type=jnp.float32)
        q = q + sense[:, s:s + 1] * gs

    rh = rh_ref[0]
    for t in range(4):
        c_ref[:, t, :] = rh[:, t:t + 1] * q

    def scatter(i, carry):
        a = pf_ref[0, 0, i]
        out_ref[a] = out_ref[a] + c_ref[i]
        return carry

    lax.fori_loop(0, C, scatter, 0, unroll=8)


def _tc_main(g, dist3, rhat3, pf3, mu2, sg2, int_w):
    return pl.pallas_call(
        _tc_main_body,
        grid=(NB,),
        in_specs=[
            pl.BlockSpec((C, NF), lambda i: (i, 0)),
            pl.BlockSpec((1, 1, C), lambda i: (i, 0, 0)),
            pl.BlockSpec((1, C, 8), lambda i: (i, 0, 0)),
            pl.BlockSpec((1, N_DIST), lambda i: (0, 0)),
            pl.BlockSpec((1, N_DIST), lambda i: (0, 0)),
            pl.BlockSpec((N_DIST, NF, NF), lambda i: (0, 0, 0)),
            pl.BlockSpec((1, 1, C), lambda i: (i, 0, 0), memory_space=pltpu.SMEM),
        ],
        out_specs=pl.BlockSpec((N_ATOMS, 8, NF), lambda i: (0, 0, 0)),
        out_shape=jax.ShapeDtypeStruct((N_ATOMS, 8, NF), jnp.float32),
        scratch_shapes=[pltpu.VMEM((C, 8, NF), jnp.float32)],
        compiler_params=pltpu.CompilerParams(
            dimension_semantics=("arbitrary",),
        ),
    )(g, dist3, rhat3, mu2, sg2, int_w, pf3)


AB = 1000


def _tc_tail_body(tf_ref, feat_ref, sw_ref, sb_ref, mw_ref, gg_ref, gb_ref, o_ref):
    inv1 = tf_ref[:, 0, :]
    inv2 = (tf_ref[:, 1, :] ** 2 + tf_ref[:, 2, :] ** 2 + tf_ref[:, 3, :] ** 2)
    acc = jnp.zeros((AB, NF), jnp.float32)
    for gidx, xg in ((0, inv1), (1, inv2)):
        m = jnp.mean(xg, axis=1, keepdims=True)
        xc = xg - m
        v = jnp.mean(xc * xc, axis=1, keepdims=True)
        xn = xc * lax.rsqrt(v + GN_EPS)
        xn = xn * gg_ref[gidx:gidx + 1, :] + gb_ref[gidx:gidx + 1, :]
        mg = mw_ref[:, gidx, :]
        acc = acc + jnp.dot(xn, mg, preferred_element_type=jnp.float32)
    selfp = lax.dot_general(feat_ref[...], sw_ref[...], (((1,), (1,)), ((), ())),
                            preferred_element_type=jnp.float32) + sb_ref[...]
    o_ref[...] = acc + selfp


def _tc_tail(tf, feat, sw, sb2, mw, gg2, gb2):
    nblk = N_ATOMS // AB
    return pl.pallas_call(
        _tc_tail_body,
        grid=(nblk,),
        in_specs=[
            pl.BlockSpec((AB, 8, NF), lambda i: (i, 0, 0)),
            pl.BlockSpec((AB, NF), lambda i: (i, 0)),
            pl.BlockSpec((NF, NF), lambda i: (0, 0)),
            pl.BlockSpec((1, NF), lambda i: (0, 0)),
            pl.BlockSpec((NF, 2, NF), lambda i: (0, 0, 0)),
            pl.BlockSpec((2, NF), lambda i: (0, 0)),
            pl.BlockSpec((2, NF), lambda i: (0, 0)),
        ],
        out_specs=pl.BlockSpec((AB, NF), lambda i: (i, 0)),
        out_shape=jax.ShapeDtypeStruct((N_ATOMS, NF), jnp.float32),
        compiler_params=pltpu.CompilerParams(
            dimension_semantics=("arbitrary",),
        ),
    )(tf, feat, sw, sb2, mw, gg2, gb2)


def kernel(in_features, pair_first, pair_second, dist_pairs, tensor_rhats,
           sense_mu, sense_sigma, int_weights, selfint_W, selfint_b,
           mixing_weights, gn_gamma, gn_beta):
    g = _sc_gather(in_features, pair_second)

    dist3 = dist_pairs.reshape(NB, 1, C)
    rhat8 = jnp.concatenate(
        [tensor_rhats, jnp.zeros((N_PAIRS, 4), jnp.float32)], axis=1)
    rhat3 = rhat8.reshape(NB, C, 8)
    pf3 = pair_first.reshape(NB, 1, C)
    mu2 = sense_mu.reshape(1, N_DIST)
    sg2 = sense_sigma.reshape(1, N_DIST)

    tf = _tc_main(g, dist3, rhat3, pf3, mu2, sg2, int_weights)

    out = _tc_tail(tf, in_features, selfint_W, selfint_b.reshape(1, NF),
                   mixing_weights, gn_gamma.reshape(2, NF),
                   gn_beta.reshape(2, NF))
    return out


# speedup vs baseline: 11.5688x; 1.0006x over previous
import functools

import jax
import jax.numpy as jnp
from jax import lax
from jax.experimental import pallas as pl
from jax.experimental.pallas import tpu as pltpu
from jax.experimental.pallas import tpu_sc as plsc

N_ATOMS = 10000
N_PAIRS = 160000
NF = 128
N_DIST = 10
HARD_CUTOFF = 5.5
GN_EPS = 1e-05

C = 1000
NB = N_PAIRS // C


def _sc_gather(table, idx):
    info = plsc.get_sparse_core_info()
    nc, ns = info.num_cores, info.num_subcores
    nw = nc * ns
    b_per_w = N_PAIRS // nw
    ch = 200
    n_ch = b_per_w // ch
    mesh = plsc.VectorSubcoreMesh(core_axis_name="c", subcore_axis_name="s")

    @functools.partial(
        pl.kernel,
        mesh=mesh,
        out_type=jax.ShapeDtypeStruct((N_PAIRS, NF), jnp.float32),
        scratch_types=[
            pltpu.VMEM((ch,), jnp.int32),
            pltpu.VMEM((ch, NF), jnp.float32),
            pltpu.SemaphoreType.DMA,
        ],
    )
    def gather_kernel(table_hbm, idx_hbm, out_hbm, idx_v, rows_v, sem):
        wid = lax.axis_index("s") * nc + lax.axis_index("c")
        base = wid * b_per_w

        def body(j, carry):
            off = base + j * ch
            pltpu.sync_copy(idx_hbm.at[pl.ds(off, ch)], idx_v)
            pltpu.async_copy(table_hbm.at[idx_v], rows_v, sem).wait()
            pltpu.sync_copy(rows_v, out_hbm.at[pl.ds(off, ch)])
            return carry

        lax.fori_loop(0, n_ch, body, 0)

    return gather_kernel(table, idx)


def _tc_main_body(g_ref, d_ref, rh_ref, mu_ref, sg_ref, w_ref, pf_ref, out_ref, c_ref):
    step = pl.program_id(0)

    @pl.when(step == 0)
    def _():
        out_ref[...] = jnp.zeros_like(out_ref)
        c_ref[...] = jnp.zeros_like(c_ref)

    gc = g_ref[...]
    d = jnp.maximum(d_ref[0], 1e-6)
    dc = d.reshape(C, 1)
    invc = 1.0 / dc
    z = (invc - mu_ref[...]) / sg_ref[...]
    base = jnp.exp(-0.5 * z * z)
    cut = jnp.where(dc < HARD_CUTOFF,
                    0.5 * (jnp.cos(jnp.pi / HARD_CUTOFF * dc) + 1.0), 0.0)
    sense = base * (cut * cut)

    q = jnp.zeros((C, NF), jnp.float32)
    for s in range(N_DIST):
        ws = w_ref[s]
        gs = lax.dot_general(gc, ws, (((1,), (1,)), ((), ())),
                             preferred_element_type=jnp.float32)
        q = q + sense[:, s:s + 1] * gs

    rh = rh_ref[0]
    for t in range(4):
        c_ref[:, t, :] = rh[:, t:t + 1] * q

    def scatter(i, carry):
        a = pf_ref[0, 0, i]
        out_ref[a] = out_ref[a] + c_ref[i]
        return carry

    lax.fori_loop(0, C, scatter, 0, unroll=16)


def _tc_main(g, dist3, rhat3, pf3, mu2, sg2, int_w):
    return pl.pallas_call(
        _tc_main_body,
        grid=(NB,),
        in_specs=[
            pl.BlockSpec((C, NF), lambda i: (i, 0)),
            pl.BlockSpec((1, 1, C), lambda i: (i, 0, 0)),
            pl.BlockSpec((1, C, 8), lambda i: (i, 0, 0)),
            pl.BlockSpec((1, N_DIST), lambda i: (0, 0)),
            pl.BlockSpec((1, N_DIST), lambda i: (0, 0)),
            pl.BlockSpec((N_DIST, NF, NF), lambda i: (0, 0, 0)),
            pl.BlockSpec((1, 1, C), lambda i: (i, 0, 0), memory_space=pltpu.SMEM),
        ],
        out_specs=pl.BlockSpec((N_ATOMS, 8, NF), lambda i: (0, 0, 0)),
        out_shape=jax.ShapeDtypeStruct((N_ATOMS, 8, NF), jnp.float32),
        scratch_shapes=[pltpu.VMEM((C, 8, NF), jnp.float32)],
        compiler_params=pltpu.CompilerParams(
            dimension_semantics=("arbitrary",),
        ),
    )(g, dist3, rhat3, mu2, sg2, int_w, pf3)


AB = 1000


def _tc_tail_body(tf_ref, feat_ref, sw_ref, sb_ref, mw_ref, gg_ref, gb_ref, o_ref):
    inv1 = tf_ref[:, 0, :]
    inv2 = (tf_ref[:, 1, :] ** 2 + tf_ref[:, 2, :] ** 2 + tf_ref[:, 3, :] ** 2)
    acc = jnp.zeros((AB, NF), jnp.float32)
    for gidx, xg in ((0, inv1), (1, inv2)):
        m = jnp.mean(xg, axis=1, keepdims=True)
        xc = xg - m
        v = jnp.mean(xc * xc, axis=1, keepdims=True)
        xn = xc * lax.rsqrt(v + GN_EPS)
        xn = xn * gg_ref[gidx:gidx + 1, :] + gb_ref[gidx:gidx + 1, :]
        mg = mw_ref[:, gidx, :]
        acc = acc + jnp.dot(xn, mg, preferred_element_type=jnp.float32)
    selfp = lax.dot_general(feat_ref[...], sw_ref[...], (((1,), (1,)), ((), ())),
                            preferred_element_type=jnp.float32) + sb_ref[...]
    o_ref[...] = acc + selfp


def _tc_tail(tf, feat, sw, sb2, mw, gg2, gb2):
    nblk = N_ATOMS // AB
    return pl.pallas_call(
        _tc_tail_body,
        grid=(nblk,),
        in_specs=[
            pl.BlockSpec((AB, 8, NF), lambda i: (i, 0, 0)),
            pl.BlockSpec((AB, NF), lambda i: (i, 0)),
            pl.BlockSpec((NF, NF), lambda i: (0, 0)),
            pl.BlockSpec((1, NF), lambda i: (0, 0)),
            pl.BlockSpec((NF, 2, NF), lambda i: (0, 0, 0)),
            pl.BlockSpec((2, NF), lambda i: (0, 0)),
            pl.BlockSpec((2, NF), lambda i: (0, 0)),
        ],
        out_specs=pl.BlockSpec((AB, NF), lambda i: (i, 0)),
        out_shape=jax.ShapeDtypeStruct((N_ATOMS, NF), jnp.float32),
        compiler_params=pltpu.CompilerParams(
            dimension_semantics=("arbitrary",),
        ),
    )(tf, feat, sw, sb2, mw, gg2, gb2)


def kernel(in_features, pair_first, pair_second, dist_pairs, tensor_rhats,
           sense_mu, sense_sigma, int_weights, selfint_W, selfint_b,
           mixing_weights, gn_gamma, gn_beta):
    g = _sc_gather(in_features, pair_second)

    dist3 = dist_pairs.reshape(NB, 1, C)
    rhat8 = jnp.concatenate(
        [tensor_rhats, jnp.zeros((N_PAIRS, 4), jnp.float32)], axis=1)
    rhat3 = rhat8.reshape(NB, C, 8)
    pf3 = pair_first.reshape(NB, 1, C)
    mu2 = sense_mu.reshape(1, N_DIST)
    sg2 = sense_sigma.reshape(1, N_DIST)

    tf = _tc_main(g, dist3, rhat3, pf3, mu2, sg2, int_weights)

    out = _tc_tail(tf, in_features, selfint_W, selfint_b.reshape(1, NF),
                   mixing_weights, gn_gamma.reshape(2, NF),
                   gn_beta.reshape(2, NF))
    return out


# speedup vs baseline: 12.1275x; 1.0483x over previous
import functools

import jax
import jax.numpy as jnp
from jax import lax
from jax.experimental import pallas as pl
from jax.experimental.pallas import tpu as pltpu
from jax.experimental.pallas import tpu_sc as plsc

N_ATOMS = 10000
N_PAIRS = 160000
NF = 128
N_DIST = 10
HARD_CUTOFF = 5.5
GN_EPS = 1e-05

C = 2000
NB = N_PAIRS // C

HALF = N_ATOMS // 2
PADROWS = 5120
DUMROWS = 1024
ROWS = PADROWS + DUMROWS
SCH = 400
PPT = N_PAIRS // 16
N_SCH = PPT // SCH
ZR = ROWS // 16
OR_ = PADROWS // 16


def _sc_gather(table, idx):
    info = plsc.get_sparse_core_info()
    nc, ns = info.num_cores, info.num_subcores
    nw = nc * ns
    b_per_w = N_PAIRS // nw
    ch = 200
    n_ch = b_per_w // ch
    mesh = plsc.VectorSubcoreMesh(core_axis_name="c", subcore_axis_name="s")

    @functools.partial(
        pl.kernel,
        mesh=mesh,
        out_type=jax.ShapeDtypeStruct((N_PAIRS, NF), jnp.float32),
        scratch_types=[
            pltpu.VMEM((ch,), jnp.int32),
            pltpu.VMEM((ch, NF), jnp.float32),
            pltpu.SemaphoreType.DMA,
        ],
    )
    def gather_kernel(table_hbm, idx_hbm, out_hbm, idx_v, rows_v, sem):
        wid = lax.axis_index("s") * nc + lax.axis_index("c")
        base = wid * b_per_w

        def body(j, carry):
            off = base + j * ch
            pltpu.sync_copy(idx_hbm.at[pl.ds(off, ch)], idx_v)
            pltpu.async_copy(table_hbm.at[idx_v], rows_v, sem).wait()
            pltpu.sync_copy(rows_v, out_hbm.at[pl.ds(off, ch)])
            return carry

        lax.fori_loop(0, n_ch, body, 0)

    return gather_kernel(table, idx)


def _tc_main_body(g_ref, d_ref, rh_ref, mu_ref, sg_ref, w_ref, c0_ref, c1_ref, c2_ref, c3_ref):
    gc = g_ref[...]
    d = jnp.maximum(d_ref[0], 1e-6)
    dc = d.reshape(C, 1)
    invc = 1.0 / dc
    z = (invc - mu_ref[...]) / sg_ref[...]
    base = jnp.exp(-0.5 * z * z)
    cut = jnp.where(dc < HARD_CUTOFF,
                    0.5 * (jnp.cos(jnp.pi / HARD_CUTOFF * dc) + 1.0), 0.0)
    sense = base * (cut * cut)

    q = jnp.zeros((C, NF), jnp.float32)
    for s in range(N_DIST):
        ws = w_ref[s]
        gs = lax.dot_general(gc, ws, (((1,), (1,)), ((), ())),
                             preferred_element_type=jnp.float32)
        q = q + sense[:, s:s + 1] * gs

    rh = rh_ref[0]
    c0_ref[...] = rh[:, 0:1] * q
    c1_ref[...] = rh[:, 1:2] * q
    c2_ref[...] = rh[:, 2:3] * q
    c3_ref[...] = rh[:, 3:4] * q


def _tc_main(g, dist3, rhat3, mu2, sg2, int_w):
    return pl.pallas_call(
        _tc_main_body,
        grid=(NB,),
        in_specs=[
            pl.BlockSpec((C, NF), lambda i: (i, 0)),
            pl.BlockSpec((1, 1, C), lambda i: (i, 0, 0)),
            pl.BlockSpec((1, C, 8), lambda i: (i, 0, 0)),
            pl.BlockSpec((1, N_DIST), lambda i: (0, 0)),
            pl.BlockSpec((1, N_DIST), lambda i: (0, 0)),
            pl.BlockSpec((N_DIST, NF, NF), lambda i: (0, 0, 0)),
        ],
        out_specs=[pl.BlockSpec((C, NF), lambda i: (i, 0))] * 4,
        out_shape=[jax.ShapeDtypeStruct((N_PAIRS, NF), jnp.float32)] * 4,
        compiler_params=pltpu.CompilerParams(
            dimension_semantics=("arbitrary",),
        ),
    )(g, dist3, rhat3, mu2, sg2, int_w)


def _sc_scatter(c0, c1, c2, c3, pf, zrows):
    mesh = plsc.VectorSubcoreMesh(core_axis_name="c", subcore_axis_name="s")

    @functools.partial(
        pl.kernel,
        mesh=mesh,
        out_type=jax.ShapeDtypeStruct((2, 4, PADROWS, NF), jnp.float32),
        scratch_types=[
            pltpu.VMEM((SCH, NF), jnp.float32),
            pltpu.VMEM((SCH,), jnp.int32),
            pltpu.VMEM((SCH,), jnp.int32),
            pltpu.VMEM_SHARED((ROWS, NF), jnp.float32),
        ],
    )
    def scatter_kernel(c0_hbm, c1_hbm, c2_hbm, c3_hbm, pf_hbm, z_hbm, out_hbm,
                       buf, pfb, idxb, acc):
        half = lax.axis_index("c")
        sid = lax.axis_index("s")
        base = sid * PPT
        lo = half * HALF
        c_refs = (c0_hbm, c1_hbm, c2_hbm, c3_hbm)
        for tp in range(4):
            pltpu.sync_copy(z_hbm, acc.at[pl.ds(sid * ZR, ZR)])
            plsc.subcore_barrier()

            def chunk(j, carry):
                off = base + j * SCH
                pltpu.sync_copy(pf_hbm.at[pl.ds(off, SCH)], pfb)
                pltpu.sync_copy(c_refs[tp].at[pl.ds(off, SCH), :], buf)

                def lane(k, carry2):
                    v = pfb[pl.ds(k * 16, 16)]
                    rel = v - lo
                    inr = (rel >= 0) & (rel < HALF)
                    dummy = PADROWS + (v & (DUMROWS - 1))
                    idxb[pl.ds(k * 16, 16)] = jnp.where(inr, rel, dummy)
                    return carry2

                lax.fori_loop(0, SCH // 16, lane, 0)
                pltpu.sync_copy(buf, acc.at[idxb], add=True)
                return carry

            lax.fori_loop(0, N_SCH, chunk, 0)
            plsc.subcore_barrier()
            pltpu.sync_copy(acc.at[pl.ds(sid * OR_, OR_)],
                            out_hbm.at[half, tp, pl.ds(sid * OR_, OR_)])
            plsc.subcore_barrier()

    return scatter_kernel(c0, c1, c2, c3, pf, zrows)


AB = HALF


def _tc_tail_body(tf_ref, feat_ref, sw_ref, sb_ref, mw_ref, gg_ref, gb_ref, o_ref):
    tfr = tf_ref[0]
    t0 = tfr[0, 0:AB, :]
    t1 = tfr[1, 0:AB, :]
    t2 = tfr[2, 0:AB, :]
    t3 = tfr[3, 0:AB, :]
    inv1 = t0
    inv2 = t1 * t1 + t2 * t2 + t3 * t3
    acc = jnp.zeros((AB, NF), jnp.float32)
    for gidx, xg in ((0, inv1), (1, inv2)):
        m = jnp.mean(xg, axis=1, keepdims=True)
        xc = xg - m
        v = jnp.mean(xc * xc, axis=1, keepdims=True)
        xn = xc * lax.rsqrt(v + GN_EPS)
        xn = xn * gg_ref[gidx:gidx + 1, :] + gb_ref[gidx:gidx + 1, :]
        mg = mw_ref[:, gidx, :]
        acc = acc + jnp.dot(xn, mg, preferred_element_type=jnp.float32)
    selfp = lax.dot_general(feat_ref[...], sw_ref[...], (((1,), (1,)), ((), ())),
                            preferred_element_type=jnp.float32) + sb_ref[...]
    o_ref[...] = acc + selfp


def _tc_tail(tfq, feat, sw, sb2, mw, gg2, gb2):
    nblk = 2
    return pl.pallas_call(
        _tc_tail_body,
        grid=(nblk,),
        in_specs=[
            pl.BlockSpec((1, 4, PADROWS, NF), lambda i: (i, 0, 0, 0)),
            pl.BlockSpec((AB, NF), lambda i: (i, 0)),
            pl.BlockSpec((NF, NF), lambda i: (0, 0)),
            pl.BlockSpec((1, NF), lambda i: (0, 0)),
            pl.BlockSpec((NF, 2, NF), lambda i: (0, 0, 0)),
            pl.BlockSpec((2, NF), lambda i: (0, 0)),
            pl.BlockSpec((2, NF), lambda i: (0, 0)),
        ],
        out_specs=pl.BlockSpec((AB, NF), lambda i: (i, 0)),
        out_shape=jax.ShapeDtypeStruct((N_ATOMS, NF), jnp.float32),
        compiler_params=pltpu.CompilerParams(
            dimension_semantics=("arbitrary",),
        ),
    )(tfq, feat, sw, sb2, mw, gg2, gb2)


def kernel(in_features, pair_first, pair_second, dist_pairs, tensor_rhats,
           sense_mu, sense_sigma, int_weights, selfint_W, selfint_b,
           mixing_weights, gn_gamma, gn_beta):
    g = _sc_gather(in_features, pair_second)

    dist3 = dist_pairs.reshape(NB, 1, C)
    rhat8 = jnp.concatenate(
        [tensor_rhats, jnp.zeros((N_PAIRS, 4), jnp.float32)], axis=1)
    rhat3 = rhat8.reshape(NB, C, 8)
    mu2 = sense_mu.reshape(1, N_DIST)
    sg2 = sense_sigma.reshape(1, N_DIST)

    c0, c1, c2, c3 = _tc_main(g, dist3, rhat3, mu2, sg2, int_weights)

    zrows = jnp.zeros((ZR, NF), jnp.float32)
    tfq = _sc_scatter(c0, c1, c2, c3, pair_first, zrows)

    out = _tc_tail(tfq, in_features, selfint_W, selfint_b.reshape(1, NF),
                   mixing_weights, gn_gamma.reshape(2, NF),
                   gn_beta.reshape(2, NF))
    return out


# speedup vs baseline: 13.6109x; 1.1223x over previous
import functools

import jax
import jax.numpy as jnp
from jax import lax
from jax.experimental import pallas as pl
from jax.experimental.pallas import tpu as pltpu
from jax.experimental.pallas import tpu_sc as plsc

N_ATOMS = 10000
N_PAIRS = 160000
NF = 128
N_DIST = 10
HARD_CUTOFF = 5.5
GN_EPS = 1e-05

C = 2000
NB = N_PAIRS // C

HALF = N_ATOMS // 2
PADROWS = 5120
DUMROWS = 1024
ROWS = PADROWS + DUMROWS
SCH = 80
PPT = N_PAIRS // 16
N_SCH = PPT // SCH
ZR = ROWS // 16
OR_ = PADROWS // 16


def _sc_gather(table, idx):
    info = plsc.get_sparse_core_info()
    nc, ns = info.num_cores, info.num_subcores
    nw = nc * ns
    b_per_w = N_PAIRS // nw
    ch = 200
    n_ch = b_per_w // ch
    mesh = plsc.VectorSubcoreMesh(core_axis_name="c", subcore_axis_name="s")

    @functools.partial(
        pl.kernel,
        mesh=mesh,
        out_type=jax.ShapeDtypeStruct((N_PAIRS, NF), jnp.float32),
        scratch_types=[
            pltpu.VMEM((ch,), jnp.int32),
            pltpu.VMEM((ch, NF), jnp.float32),
            pltpu.SemaphoreType.DMA,
        ],
    )
    def gather_kernel(table_hbm, idx_hbm, out_hbm, idx_v, rows_v, sem):
        wid = lax.axis_index("s") * nc + lax.axis_index("c")
        base = wid * b_per_w

        def body(j, carry):
            off = base + j * ch
            pltpu.sync_copy(idx_hbm.at[pl.ds(off, ch)], idx_v)
            pltpu.async_copy(table_hbm.at[idx_v], rows_v, sem).wait()
            pltpu.sync_copy(rows_v, out_hbm.at[pl.ds(off, ch)])
            return carry

        lax.fori_loop(0, n_ch, body, 0)

    return gather_kernel(table, idx)


def _tc_main_body(g_ref, d_ref, rh_ref, mu_ref, sg_ref, w_ref, c0_ref, c1_ref, c2_ref, c3_ref):
    gc = g_ref[...]
    d = jnp.maximum(d_ref[0], 1e-6)
    dc = d.reshape(C, 1)
    invc = 1.0 / dc
    z = (invc - mu_ref[...]) / sg_ref[...]
    base = jnp.exp(-0.5 * z * z)
    cut = jnp.where(dc < HARD_CUTOFF,
                    0.5 * (jnp.cos(jnp.pi / HARD_CUTOFF * dc) + 1.0), 0.0)
    sense = base * (cut * cut)

    gb = gc.astype(jnp.bfloat16)
    wb = w_ref[...].astype(jnp.bfloat16)
    q = jnp.zeros((C, NF), jnp.float32)
    for s in range(N_DIST):
        gs = lax.dot_general(gb, wb[s], (((1,), (1,)), ((), ())),
                             preferred_element_type=jnp.float32)
        q = q + sense[:, s:s + 1] * gs

    rh = rh_ref[0]
    c0_ref[...] = rh[:, 0:1] * q
    c1_ref[...] = rh[:, 1:2] * q
    c2_ref[...] = rh[:, 2:3] * q
    c3_ref[...] = rh[:, 3:4] * q


def _tc_main(g, dist3, rhat3, mu2, sg2, int_w):
    return pl.pallas_call(
        _tc_main_body,
        grid=(NB,),
        in_specs=[
            pl.BlockSpec((C, NF), lambda i: (i, 0)),
            pl.BlockSpec((1, 1, C), lambda i: (i, 0, 0)),
            pl.BlockSpec((1, C, 8), lambda i: (i, 0, 0)),
            pl.BlockSpec((1, N_DIST), lambda i: (0, 0)),
            pl.BlockSpec((1, N_DIST), lambda i: (0, 0)),
            pl.BlockSpec((N_DIST, NF, NF), lambda i: (0, 0, 0)),
        ],
        out_specs=[pl.BlockSpec((C, NF), lambda i: (i, 0))] * 4,
        out_shape=[jax.ShapeDtypeStruct((N_PAIRS, NF), jnp.float32)] * 4,
        compiler_params=pltpu.CompilerParams(
            dimension_semantics=("arbitrary",),
        ),
    )(g, dist3, rhat3, mu2, sg2, int_w)


def _sc_scatter(c0, c1, c2, c3, pf, zrows):
    mesh = plsc.VectorSubcoreMesh(core_axis_name="c", subcore_axis_name="s")

    @functools.partial(
        pl.kernel,
        mesh=mesh,
        out_type=jax.ShapeDtypeStruct((2, 4, PADROWS, NF), jnp.float32),
        scratch_types=[
            pltpu.VMEM((SCH, NF), jnp.float32),
            pltpu.VMEM((SCH, NF), jnp.float32),
            pltpu.VMEM((PPT,), jnp.int32),
            pltpu.VMEM((N_SCH, 1, SCH), jnp.int32),
            pltpu.VMEM_SHARED((ROWS, NF), jnp.float32),
            pltpu.SemaphoreType.DMA,
            pltpu.SemaphoreType.DMA,
        ],
    )
    def scatter_kernel(c0_hbm, c1_hbm, c2_hbm, c3_hbm, pf_hbm, z_hbm, out_hbm,
                       buf_a, buf_b, pfb, idx2, acc, sem_a, sem_b):
        half = lax.axis_index("c")
        sid = lax.axis_index("s")
        base = sid * PPT
        lo = half * HALF
        c_refs = (c0_hbm, c1_hbm, c2_hbm, c3_hbm)

        pltpu.sync_copy(pf_hbm.at[pl.ds(base, PPT)], pfb)

        def prep(j, carry):
            for kk in range(SCH // 16):
                v = pfb[pl.ds(j * SCH + kk * 16, 16)]
                rel = v - lo
                inr = (rel >= 0) & (rel < HALF)
                dummy = PADROWS + (v & (DUMROWS - 1))
                idx2[j, 0, pl.ds(kk * 16, 16)] = jnp.where(inr, rel, dummy)
            return carry

        lax.fori_loop(0, N_SCH, prep, 0)

        for tp in range(4):
            pltpu.sync_copy(z_hbm, acc.at[pl.ds(sid * ZR, ZR)])
            plsc.subcore_barrier()
            src = c_refs[tp]

            pltpu.async_copy(src.at[pl.ds(base, SCH), :], buf_a, sem_a)

            def two(jj, carry):
                j1 = 2 * jj + 1
                pltpu.async_copy(src.at[pl.ds(base + j1 * SCH, SCH), :],
                                 buf_b, sem_b)
                pltpu.make_async_copy(src.at[pl.ds(base, SCH), :],
                                      buf_a, sem_a).wait()
                pltpu.sync_copy(buf_a, acc.at[idx2.at[j1 - 1, 0]], add=True)
                j2 = 2 * jj + 2
                pltpu.async_copy(src.at[pl.ds(base + j2 * SCH, SCH), :],
                                 buf_a, sem_a)
                pltpu.make_async_copy(src.at[pl.ds(base, SCH), :],
                                      buf_b, sem_b).wait()
                pltpu.sync_copy(buf_b, acc.at[idx2.at[j1, 0]], add=True)
                return carry

            lax.fori_loop(0, (N_SCH - 1) // 2, two, 0)
            pltpu.make_async_copy(src.at[pl.ds(base, SCH), :],
                                  buf_a, sem_a).wait()
            pltpu.sync_copy(buf_a, acc.at[idx2.at[N_SCH - 1, 0]], add=True)

            plsc.subcore_barrier()
            pltpu.sync_copy(acc.at[pl.ds(sid * OR_, OR_)],
                            out_hbm.at[half, tp, pl.ds(sid * OR_, OR_)])
            plsc.subcore_barrier()

    return scatter_kernel(c0, c1, c2, c3, pf, zrows)


AB = HALF


def _tc_tail_body(tf_ref, feat_ref, sw_ref, sb_ref, mw_ref, gg_ref, gb_ref, o_ref):
    tfr = tf_ref[0]
    t0 = tfr[0, 0:AB, :]
    t1 = tfr[1, 0:AB, :]
    t2 = tfr[2, 0:AB, :]
    t3 = tfr[3, 0:AB, :]
    inv1 = t0
    inv2 = t1 * t1 + t2 * t2 + t3 * t3
    acc = jnp.zeros((AB, NF), jnp.float32)
    for gidx, xg in ((0, inv1), (1, inv2)):
        m = jnp.mean(xg, axis=1, keepdims=True)
        xc = xg - m
        v = jnp.mean(xc * xc, axis=1, keepdims=True)
        xn = xc * lax.rsqrt(v + GN_EPS)
        xn = xn * gg_ref[gidx:gidx + 1, :] + gb_ref[gidx:gidx + 1, :]
        mg = mw_ref[:, gidx, :]
        acc = acc + jnp.dot(xn, mg, preferred_element_type=jnp.float32)
    selfp = lax.dot_general(feat_ref[...], sw_ref[...], (((1,), (1,)), ((), ())),
                            preferred_element_type=jnp.float32) + sb_ref[...]
    o_ref[...] = acc + selfp


def _tc_tail(tfq, feat, sw, sb2, mw, gg2, gb2):
    nblk = 2
    return pl.pallas_call(
        _tc_tail_body,
        grid=(nblk,),
        in_specs=[
            pl.BlockSpec((1, 4, PADROWS, NF), lambda i: (i, 0, 0, 0)),
            pl.BlockSpec((AB, NF), lambda i: (i, 0)),
            pl.BlockSpec((NF, NF), lambda i: (0, 0)),
            pl.BlockSpec((1, NF), lambda i: (0, 0)),
            pl.BlockSpec((NF, 2, NF), lambda i: (0, 0, 0)),
            pl.BlockSpec((2, NF), lambda i: (0, 0)),
            pl.BlockSpec((2, NF), lambda i: (0, 0)),
        ],
        out_specs=pl.BlockSpec((AB, NF), lambda i: (i, 0)),
        out_shape=jax.ShapeDtypeStruct((N_ATOMS, NF), jnp.float32),
        compiler_params=pltpu.CompilerParams(
            dimension_semantics=("arbitrary",),
        ),
    )(tfq, feat, sw, sb2, mw, gg2, gb2)


def kernel(in_features, pair_first, pair_second, dist_pairs, tensor_rhats,
           sense_mu, sense_sigma, int_weights, selfint_W, selfint_b,
           mixing_weights, gn_gamma, gn_beta):
    g = _sc_gather(in_features, pair_second)

    dist3 = dist_pairs.reshape(NB, 1, C)
    rhat8 = jnp.concatenate(
        [tensor_rhats, jnp.zeros((N_PAIRS, 4), jnp.float32)], axis=1)
    rhat3 = rhat8.reshape(NB, C, 8)
    mu2 = sense_mu.reshape(1, N_DIST)
    sg2 = sense_sigma.reshape(1, N_DIST)

    c0, c1, c2, c3 = _tc_main(g, dist3, rhat3, mu2, sg2, int_weights)

    zrows = jnp.zeros((ZR, NF), jnp.float32)
    tfq = _sc_scatter(c0, c1, c2, c3, pair_first, zrows)

    out = _tc_tail(tfq, in_features, selfint_W, selfint_b.reshape(1, NF),
                   mixing_weights, gn_gamma.reshape(2, NF),
                   gn_beta.reshape(2, NF))
    return out


# speedup vs baseline: 16.9096x; 1.2424x over previous
import functools

import jax
import jax.numpy as jnp
from jax import lax
from jax.experimental import pallas as pl
from jax.experimental.pallas import tpu as pltpu
from jax.experimental.pallas import tpu_sc as plsc

N_ATOMS = 10000
N_PAIRS = 160000
NF = 128
N_DIST = 10
HARD_CUTOFF = 5.5
GN_EPS = 1e-05

C = 2000
NB = N_PAIRS // C

HALF = N_ATOMS // 2
PADROWS = 5120
DUMROWS = 1024
ROWS = PADROWS + DUMROWS
SCH = 80
PPT = N_PAIRS // 16
N_SCH = PPT // SCH
ZR = ROWS // 16
OR_ = PADROWS // 16


def _sc_gather(table, idx):
    info = plsc.get_sparse_core_info()
    nc, ns = info.num_cores, info.num_subcores
    nw = nc * ns
    b_per_w = N_PAIRS // nw
    ch = 200
    n_ch = b_per_w // ch
    mesh = plsc.VectorSubcoreMesh(core_axis_name="c", subcore_axis_name="s")

    @functools.partial(
        pl.kernel,
        mesh=mesh,
        out_type=jax.ShapeDtypeStruct((N_PAIRS, NF), jnp.float32),
        scratch_types=[
            pltpu.VMEM((ch,), jnp.int32),
            pltpu.VMEM((ch, NF), jnp.float32),
            pltpu.SemaphoreType.DMA,
        ],
    )
    def gather_kernel(table_hbm, idx_hbm, out_hbm, idx_v, rows_v, sem):
        wid = lax.axis_index("s") * nc + lax.axis_index("c")
        base = wid * b_per_w

        def body(j, carry):
            off = base + j * ch
            pltpu.sync_copy(idx_hbm.at[pl.ds(off, ch)], idx_v)
            pltpu.async_copy(table_hbm.at[idx_v], rows_v, sem).wait()
            pltpu.sync_copy(rows_v, out_hbm.at[pl.ds(off, ch)])
            return carry

        lax.fori_loop(0, n_ch, body, 0)

    return gather_kernel(table, idx)


def _tc_main_body(g_ref, d_ref, rh_ref, mu_ref, sg_ref, w_ref, c0_ref, c1_ref, c2_ref, c3_ref):
    gc = g_ref[...]
    d_row = jnp.maximum(d_ref[0], 1e-6)
    inv_row = 1.0 / d_row
    cut_row = jnp.where(d_row < HARD_CUTOFF,
                        0.5 * (jnp.cos(jnp.pi / HARD_CUTOFF * d_row) + 1.0), 0.0)
    cutsq = (cut_row * cut_row).reshape(C, 1)
    invc = inv_row.reshape(C, 1)
    z = (invc - mu_ref[...]) / sg_ref[...]
    sense = jnp.exp(-0.5 * z * z)

    gb = gc.astype(jnp.bfloat16)
    wb = w_ref[...].astype(jnp.bfloat16)
    q = jnp.zeros((C, NF), jnp.float32)
    for s in range(N_DIST):
        gs = lax.dot_general(gb, wb[s], (((1,), (1,)), ((), ())),
                             preferred_element_type=jnp.float32)
        q = q + sense[:, s:s + 1] * gs

    rh = rh_ref[0] * cutsq
    c0_ref[...] = rh[:, 0:1] * q
    c1_ref[...] = rh[:, 1:2] * q
    c2_ref[...] = rh[:, 2:3] * q
    c3_ref[...] = rh[:, 3:4] * q


def _tc_main(g, dist3, rhat3, mu2, sg2, int_w):
    return pl.pallas_call(
        _tc_main_body,
        grid=(NB,),
        in_specs=[
            pl.BlockSpec((C, NF), lambda i: (i, 0)),
            pl.BlockSpec((1, 1, C), lambda i: (i, 0, 0)),
            pl.BlockSpec((1, C, 8), lambda i: (i, 0, 0)),
            pl.BlockSpec((1, N_DIST), lambda i: (0, 0)),
            pl.BlockSpec((1, N_DIST), lambda i: (0, 0)),
            pl.BlockSpec((N_DIST, NF, NF), lambda i: (0, 0, 0)),
        ],
        out_specs=[pl.BlockSpec((C, NF), lambda i: (i, 0))] * 4,
        out_shape=[jax.ShapeDtypeStruct((N_PAIRS, NF), jnp.float32)] * 4,
        compiler_params=pltpu.CompilerParams(
            dimension_semantics=("arbitrary",),
        ),
    )(g, dist3, rhat3, mu2, sg2, int_w)


def _sc_scatter(c0, c1, c2, c3, pf, zrows):
    mesh = plsc.VectorSubcoreMesh(core_axis_name="c", subcore_axis_name="s")

    @functools.partial(
        pl.kernel,
        mesh=mesh,
        out_type=jax.ShapeDtypeStruct((2, 4, PADROWS, NF), jnp.float32),
        scratch_types=[
            pltpu.VMEM((SCH, NF), jnp.float32),
            pltpu.VMEM((SCH, NF), jnp.float32),
            pltpu.VMEM((PPT,), jnp.int32),
            pltpu.VMEM((N_SCH, 1, SCH), jnp.int32),
            pltpu.VMEM_SHARED((ROWS, NF), jnp.float32),
            pltpu.SemaphoreType.DMA,
            pltpu.SemaphoreType.DMA,
        ],
    )
    def scatter_kernel(c0_hbm, c1_hbm, c2_hbm, c3_hbm, pf_hbm, z_hbm, out_hbm,
                       buf_a, buf_b, pfb, idx2, acc, sem_a, sem_b):
        half = lax.axis_index("c")
        sid = lax.axis_index("s")
        base = sid * PPT
        lo = half * HALF
        c_refs = (c0_hbm, c1_hbm, c2_hbm, c3_hbm)

        pltpu.sync_copy(pf_hbm.at[pl.ds(base, PPT)], pfb)

        def prep(j, carry):
            for kk in range(SCH // 16):
                v = pfb[pl.ds(j * SCH + kk * 16, 16)]
                rel = v - lo
                inr = (rel >= 0) & (rel < HALF)
                dummy = PADROWS + (v & (DUMROWS - 1))
                idx2[j, 0, pl.ds(kk * 16, 16)] = jnp.where(inr, rel, dummy)
            return carry

        lax.fori_loop(0, N_SCH, prep, 0)

        for tp in range(4):
            pltpu.sync_copy(z_hbm, acc.at[pl.ds(sid * ZR, ZR)])
            plsc.subcore_barrier()
            src = c_refs[tp]

            pltpu.async_copy(src.at[pl.ds(base, SCH), :], buf_a, sem_a)

            def two(jj, carry):
                j1 = 2 * jj + 1
                pltpu.async_copy(src.at[pl.ds(base + j1 * SCH, SCH), :],
                                 buf_b, sem_b)
                pltpu.make_async_copy(src.at[pl.ds(base, SCH), :],
                                      buf_a, sem_a).wait()
                pltpu.sync_copy(buf_a, acc.at[idx2.at[j1 - 1, 0]], add=True)
                j2 = 2 * jj + 2
                pltpu.async_copy(src.at[pl.ds(base + j2 * SCH, SCH), :],
                                 buf_a, sem_a)
                pltpu.make_async_copy(src.at[pl.ds(base, SCH), :],
                                      buf_b, sem_b).wait()
                pltpu.sync_copy(buf_b, acc.at[idx2.at[j1, 0]], add=True)
                return carry

            lax.fori_loop(0, (N_SCH - 1) // 2, two, 0)
            pltpu.make_async_copy(src.at[pl.ds(base, SCH), :],
                                  buf_a, sem_a).wait()
            pltpu.sync_copy(buf_a, acc.at[idx2.at[N_SCH - 1, 0]], add=True)

            plsc.subcore_barrier()
            pltpu.sync_copy(acc.at[pl.ds(sid * OR_, OR_)],
                            out_hbm.at[half, tp, pl.ds(sid * OR_, OR_)])
            plsc.subcore_barrier()

    return scatter_kernel(c0, c1, c2, c3, pf, zrows)


AB = HALF


def _tc_tail_body(tf_ref, feat_ref, sw_ref, sb_ref, mw_ref, gg_ref, gb_ref, o_ref):
    tfr = tf_ref[0]
    t0 = tfr[0, 0:AB, :]
    t1 = tfr[1, 0:AB, :]
    t2 = tfr[2, 0:AB, :]
    t3 = tfr[3, 0:AB, :]
    inv1 = t0
    inv2 = t1 * t1 + t2 * t2 + t3 * t3
    acc = jnp.zeros((AB, NF), jnp.float32)
    for gidx, xg in ((0, inv1), (1, inv2)):
        m = jnp.mean(xg, axis=1, keepdims=True)
        xc = xg - m
        v = jnp.mean(xc * xc, axis=1, keepdims=True)
        xn = xc * lax.rsqrt(v + GN_EPS)
        xn = xn * gg_ref[gidx:gidx + 1, :] + gb_ref[gidx:gidx + 1, :]
        mg = mw_ref[:, gidx, :]
        acc = acc + jnp.dot(xn, mg, preferred_element_type=jnp.float32)
    selfp = lax.dot_general(feat_ref[...], sw_ref[...], (((1,), (1,)), ((), ())),
                            preferred_element_type=jnp.float32) + sb_ref[...]
    o_ref[...] = acc + selfp


def _tc_tail(tfq, feat, sw, sb2, mw, gg2, gb2):
    nblk = 2
    return pl.pallas_call(
        _tc_tail_body,
        grid=(nblk,),
        in_specs=[
            pl.BlockSpec((1, 4, PADROWS, NF), lambda i: (i, 0, 0, 0)),
            pl.BlockSpec((AB, NF), lambda i: (i, 0)),
            pl.BlockSpec((NF, NF), lambda i: (0, 0)),
            pl.BlockSpec((1, NF), lambda i: (0, 0)),
            pl.BlockSpec((NF, 2, NF), lambda i: (0, 0, 0)),
            pl.BlockSpec((2, NF), lambda i: (0, 0)),
            pl.BlockSpec((2, NF), lambda i: (0, 0)),
        ],
        out_specs=pl.BlockSpec((AB, NF), lambda i: (i, 0)),
        out_shape=jax.ShapeDtypeStruct((N_ATOMS, NF), jnp.float32),
        compiler_params=pltpu.CompilerParams(
            dimension_semantics=("arbitrary",),
        ),
    )(tfq, feat, sw, sb2, mw, gg2, gb2)


def kernel(in_features, pair_first, pair_second, dist_pairs, tensor_rhats,
           sense_mu, sense_sigma, int_weights, selfint_W, selfint_b,
           mixing_weights, gn_gamma, gn_beta):
    g = _sc_gather(in_features, pair_second)

    dist3 = dist_pairs.reshape(NB, 1, C)
    rhat8 = jnp.concatenate(
        [tensor_rhats, jnp.zeros((N_PAIRS, 4), jnp.float32)], axis=1)
    rhat3 = rhat8.reshape(NB, C, 8)
    mu2 = sense_mu.reshape(1, N_DIST)
    sg2 = sense_sigma.reshape(1, N_DIST)

    c0, c1, c2, c3 = _tc_main(g, dist3, rhat3, mu2, sg2, int_weights)

    zrows = jnp.zeros((ZR, NF), jnp.float32)
    tfq = _sc_scatter(c0, c1, c2, c3, pair_first, zrows)

    out = _tc_tail(tfq, in_features, selfint_W, selfint_b.reshape(1, NF),
                   mixing_weights, gn_gamma.reshape(2, NF),
                   gn_beta.reshape(2, NF))
    return out


# speedup vs baseline: 19.8283x; 1.1726x over previous
import functools

import jax
import jax.numpy as jnp
from jax import lax
from jax.experimental import pallas as pl
from jax.experimental.pallas import tpu as pltpu
from jax.experimental.pallas import tpu_sc as plsc

N_ATOMS = 10000
N_PAIRS = 160000
NF = 128
N_DIST = 10
HARD_CUTOFF = 5.5
GN_EPS = 1e-05

C = 2000
NB = N_PAIRS // C

HALF = N_ATOMS // 2
PADROWS = 5120
DUMROWS = 1024
ROWS = PADROWS + DUMROWS
SCH = 80
PPT = N_PAIRS // 16
N_SCH = PPT // SCH
ZR = ROWS // 16
OR_ = PADROWS // 16


def _sc_gather(table, idx):
    info = plsc.get_sparse_core_info()
    nc, ns = info.num_cores, info.num_subcores
    nw = nc * ns
    b_per_w = N_PAIRS // nw
    ch = 200
    n_ch = b_per_w // ch
    mesh = plsc.VectorSubcoreMesh(core_axis_name="c", subcore_axis_name="s")

    @functools.partial(
        pl.kernel,
        mesh=mesh,
        out_type=jax.ShapeDtypeStruct((N_PAIRS, NF), jnp.float32),
        scratch_types=[
            pltpu.VMEM((ch,), jnp.int32),
            pltpu.VMEM((ch, NF), jnp.float32),
            pltpu.SemaphoreType.DMA,
        ],
    )
    def gather_kernel(table_hbm, idx_hbm, out_hbm, idx_v, rows_v, sem):
        wid = lax.axis_index("s") * nc + lax.axis_index("c")
        base = wid * b_per_w

        def body(j, carry):
            off = base + j * ch
            pltpu.sync_copy(idx_hbm.at[pl.ds(off, ch)], idx_v)
            pltpu.async_copy(table_hbm.at[idx_v], rows_v, sem).wait()
            pltpu.sync_copy(rows_v, out_hbm.at[pl.ds(off, ch)])
            return carry

        lax.fori_loop(0, n_ch, body, 0)

    return gather_kernel(table, idx)


def _tc_main_body(g_ref, d_ref, rh_ref, mu_ref, sg_ref, w_ref, b_ref, c0_ref, c1_ref, c2_ref, c3_ref):
    gc = g_ref[...]
    d_row = jnp.maximum(d_ref[0], 1e-6)
    inv_row = 1.0 / d_row
    cut_row = jnp.where(d_row < HARD_CUTOFF,
                        0.5 * (jnp.cos(jnp.pi / HARD_CUTOFF * d_row) + 1.0), 0.0)
    cutsq = (cut_row * cut_row).reshape(C, 1)
    invc = inv_row.reshape(C, 1)
    z = (invc - mu_ref[...]) / sg_ref[...]
    sense = jnp.exp(-0.5 * z * z)

    gb = gc.astype(jnp.bfloat16)
    gs_all = jnp.dot(gb, w_ref[...], preferred_element_type=jnp.float32)
    e_all = jnp.dot(sense.astype(jnp.bfloat16), b_ref[...],
                    preferred_element_type=jnp.float32)
    prod = e_all * gs_all
    q = prod[:, 0:NF]
    for s in range(1, N_DIST):
        q = q + prod[:, s * NF:(s + 1) * NF]

    rh = rh_ref[0] * cutsq
    c0_ref[...] = rh[:, 0:1] * q
    c1_ref[...] = rh[:, 1:2] * q
    c2_ref[...] = rh[:, 2:3] * q
    c3_ref[...] = rh[:, 3:4] * q


def _tc_main(g, dist3, rhat3, mu2, sg2, wcat, bsel):
    return pl.pallas_call(
        _tc_main_body,
        grid=(NB,),
        in_specs=[
            pl.BlockSpec((C, NF), lambda i: (i, 0)),
            pl.BlockSpec((1, 1, C), lambda i: (i, 0, 0)),
            pl.BlockSpec((1, C, 8), lambda i: (i, 0, 0)),
            pl.BlockSpec((1, N_DIST), lambda i: (0, 0)),
            pl.BlockSpec((1, N_DIST), lambda i: (0, 0)),
            pl.BlockSpec((NF, N_DIST * NF), lambda i: (0, 0)),
            pl.BlockSpec((N_DIST, N_DIST * NF), lambda i: (0, 0)),
        ],
        out_specs=[pl.BlockSpec((C, NF), lambda i: (i, 0))] * 4,
        out_shape=[jax.ShapeDtypeStruct((N_PAIRS, NF), jnp.float32)] * 4,
        compiler_params=pltpu.CompilerParams(
            dimension_semantics=("arbitrary",),
        ),
    )(g, dist3, rhat3, mu2, sg2, wcat, bsel)


def _sc_scatter(c0, c1, c2, c3, pf, zrows):
    mesh = plsc.VectorSubcoreMesh(core_axis_name="c", subcore_axis_name="s")

    @functools.partial(
        pl.kernel,
        mesh=mesh,
        out_type=jax.ShapeDtypeStruct((2, 4, PADROWS, NF), jnp.float32),
        scratch_types=[
            pltpu.VMEM((SCH, NF), jnp.float32),
            pltpu.VMEM((SCH, NF), jnp.float32),
            pltpu.VMEM((PPT,), jnp.int32),
            pltpu.VMEM((N_SCH, 1, SCH), jnp.int32),
            pltpu.VMEM_SHARED((ROWS, NF), jnp.float32),
            pltpu.SemaphoreType.DMA,
            pltpu.SemaphoreType.DMA,
        ],
    )
    def scatter_kernel(c0_hbm, c1_hbm, c2_hbm, c3_hbm, pf_hbm, z_hbm, out_hbm,
                       buf_a, buf_b, pfb, idx2, acc, sem_a, sem_b):
        half = lax.axis_index("c")
        sid = lax.axis_index("s")
        base = sid * PPT
        lo = half * HALF
        c_refs = (c0_hbm, c1_hbm, c2_hbm, c3_hbm)

        pltpu.sync_copy(pf_hbm.at[pl.ds(base, PPT)], pfb)

        def prep(j, carry):
            for kk in range(SCH // 16):
                v = pfb[pl.ds(j * SCH + kk * 16, 16)]
                rel = v - lo
                inr = (rel >= 0) & (rel < HALF)
                dummy = PADROWS + (v & (DUMROWS - 1))
                idx2[j, 0, pl.ds(kk * 16, 16)] = jnp.where(inr, rel, dummy)
            return carry

        lax.fori_loop(0, N_SCH, prep, 0)

        for tp in range(4):
            pltpu.sync_copy(z_hbm, acc.at[pl.ds(sid * ZR, ZR)])
            plsc.subcore_barrier()
            src = c_refs[tp]

            pltpu.async_copy(src.at[pl.ds(base, SCH), :], buf_a, sem_a)

            def two(jj, carry):
                j1 = 2 * jj + 1
                pltpu.async_copy(src.at[pl.ds(base + j1 * SCH, SCH), :],
                                 buf_b, sem_b)
                pltpu.make_async_copy(src.at[pl.ds(base, SCH), :],
                                      buf_a, sem_a).wait()
                pltpu.sync_copy(buf_a, acc.at[idx2.at[j1 - 1, 0]], add=True)
                j2 = 2 * jj + 2
                pltpu.async_copy(src.at[pl.ds(base + j2 * SCH, SCH), :],
                                 buf_a, sem_a)
                pltpu.make_async_copy(src.at[pl.ds(base, SCH), :],
                                      buf_b, sem_b).wait()
                pltpu.sync_copy(buf_b, acc.at[idx2.at[j1, 0]], add=True)
                return carry

            lax.fori_loop(0, (N_SCH - 1) // 2, two, 0)
            pltpu.make_async_copy(src.at[pl.ds(base, SCH), :],
                                  buf_a, sem_a).wait()
            pltpu.sync_copy(buf_a, acc.at[idx2.at[N_SCH - 1, 0]], add=True)

            plsc.subcore_barrier()
            pltpu.sync_copy(acc.at[pl.ds(sid * OR_, OR_)],
                            out_hbm.at[half, tp, pl.ds(sid * OR_, OR_)])
            plsc.subcore_barrier()

    return scatter_kernel(c0, c1, c2, c3, pf, zrows)


AB = HALF


def _tc_tail_body(tf_ref, feat_ref, sw_ref, sb_ref, mw_ref, gg_ref, gb_ref, o_ref):
    tfr = tf_ref[0]
    t0 = tfr[0, 0:AB, :]
    t1 = tfr[1, 0:AB, :]
    t2 = tfr[2, 0:AB, :]
    t3 = tfr[3, 0:AB, :]
    inv1 = t0
    inv2 = t1 * t1 + t2 * t2 + t3 * t3
    acc = jnp.zeros((AB, NF), jnp.float32)
    for gidx, xg in ((0, inv1), (1, inv2)):
        m = jnp.mean(xg, axis=1, keepdims=True)
        xc = xg - m
        v = jnp.mean(xc * xc, axis=1, keepdims=True)
        xn = xc * lax.rsqrt(v + GN_EPS)
        xn = xn * gg_ref[gidx:gidx + 1, :] + gb_ref[gidx:gidx + 1, :]
        mg = mw_ref[:, gidx, :]
        acc = acc + jnp.dot(xn, mg, preferred_element_type=jnp.float32)
    selfp = lax.dot_general(feat_ref[...], sw_ref[...], (((1,), (1,)), ((), ())),
                            preferred_element_type=jnp.float32) + sb_ref[...]
    o_ref[...] = acc + selfp


def _tc_tail(tfq, feat, sw, sb2, mw, gg2, gb2):
    nblk = 2
    return pl.pallas_call(
        _tc_tail_body,
        grid=(nblk,),
        in_specs=[
            pl.BlockSpec((1, 4, PADROWS, NF), lambda i: (i, 0, 0, 0)),
            pl.BlockSpec((AB, NF), lambda i: (i, 0)),
            pl.BlockSpec((NF, NF), lambda i: (0, 0)),
            pl.BlockSpec((1, NF), lambda i: (0, 0)),
            pl.BlockSpec((NF, 2, NF), lambda i: (0, 0, 0)),
            pl.BlockSpec((2, NF), lambda i: (0, 0)),
            pl.BlockSpec((2, NF), lambda i: (0, 0)),
        ],
        out_specs=pl.BlockSpec((AB, NF), lambda i: (i, 0)),
        out_shape=jax.ShapeDtypeStruct((N_ATOMS, NF), jnp.float32),
        compiler_params=pltpu.CompilerParams(
            dimension_semantics=("arbitrary",),
        ),
    )(tfq, feat, sw, sb2, mw, gg2, gb2)


def kernel(in_features, pair_first, pair_second, dist_pairs, tensor_rhats,
           sense_mu, sense_sigma, int_weights, selfint_W, selfint_b,
           mixing_weights, gn_gamma, gn_beta):
    g = _sc_gather(in_features, pair_second)

    dist3 = dist_pairs.reshape(NB, 1, C)
    rhat8 = jnp.concatenate(
        [tensor_rhats, jnp.zeros((N_PAIRS, 4), jnp.float32)], axis=1)
    rhat3 = rhat8.reshape(NB, C, 8)
    mu2 = sense_mu.reshape(1, N_DIST)
    sg2 = sense_sigma.reshape(1, N_DIST)

    wcat = int_weights.transpose(2, 0, 1).reshape(NF, N_DIST * NF).astype(jnp.bfloat16)
    bsel = (jnp.arange(N_DIST * NF)[None, :] // NF ==
            jnp.arange(N_DIST)[:, None]).astype(jnp.bfloat16)
    c0, c1, c2, c3 = _tc_main(g, dist3, rhat3, mu2, sg2, wcat, bsel)

    zrows = jnp.zeros((ZR, NF), jnp.float32)
    tfq = _sc_scatter(c0, c1, c2, c3, pair_first, zrows)

    out = _tc_tail(tfq, in_features, selfint_W, selfint_b.reshape(1, NF),
                   mixing_weights, gn_gamma.reshape(2, NF),
                   gn_beta.reshape(2, NF))
    return out


# speedup vs baseline: 24.5067x; 1.2359x over previous
import functools

import jax
import jax.numpy as jnp
from jax import lax
from jax.experimental import pallas as pl
from jax.experimental.pallas import tpu as pltpu
from jax.experimental.pallas import tpu_sc as plsc

N_ATOMS = 10000
N_PAIRS = 160000
NF = 128
N_DIST = 10
HARD_CUTOFF = 5.5
GN_EPS = 1e-05

C = 2000
NB = N_PAIRS // C

P_PAD = 160640
PPH = N_PAIRS // 2
PPT = PPH // 16
SCH = 80
N_SCH = 63
PFB = 5056
PADROWS = 10112
DUMROWS = 128
ROWS = PADROWS + DUMROWS
ZR = ROWS // 16
OR_ = PADROWS // 16


def _sc_gather(table, idx):
    info = plsc.get_sparse_core_info()
    nc, ns = info.num_cores, info.num_subcores
    nw = nc * ns
    b_per_w = N_PAIRS // nw
    ch = 200
    n_ch = b_per_w // ch
    mesh = plsc.VectorSubcoreMesh(core_axis_name="c", subcore_axis_name="s")

    @functools.partial(
        pl.kernel,
        mesh=mesh,
        out_type=jax.ShapeDtypeStruct((N_PAIRS, NF), jnp.float32),
        scratch_types=[
            pltpu.VMEM((ch,), jnp.int32),
            pltpu.VMEM((ch, NF), jnp.float32),
            pltpu.SemaphoreType.DMA,
        ],
    )
    def gather_kernel(table_hbm, idx_hbm, out_hbm, idx_v, rows_v, sem):
        wid = lax.axis_index("s") * nc + lax.axis_index("c")
        base = wid * b_per_w

        def body(j, carry):
            off = base + j * ch
            pltpu.sync_copy(idx_hbm.at[pl.ds(off, ch)], idx_v)
            pltpu.async_copy(table_hbm.at[idx_v], rows_v, sem).wait()
            pltpu.sync_copy(rows_v, out_hbm.at[pl.ds(off, ch)])
            return carry

        lax.fori_loop(0, n_ch, body, 0)

    return gather_kernel(table, idx)


def _tc_main_body(g_ref, d_ref, rh_ref, mu_ref, sg_ref, w_ref, b_ref, c0_ref, c1_ref, c2_ref, c3_ref):
    gc = g_ref[...]
    d_row = jnp.maximum(d_ref[0], 1e-6)
    inv_row = 1.0 / d_row
    cut_row = jnp.where(d_row < HARD_CUTOFF,
                        0.5 * (jnp.cos(jnp.pi / HARD_CUTOFF * d_row) + 1.0), 0.0)
    cutsq = (cut_row * cut_row).reshape(C, 1)
    invc = inv_row.reshape(C, 1)
    z = (invc - mu_ref[...]) / sg_ref[...]
    sense = jnp.exp(-0.5 * z * z)

    gb = gc.astype(jnp.bfloat16)
    gs_all = jnp.dot(gb, w_ref[...], preferred_element_type=jnp.float32)
    e_all = jnp.dot(sense.astype(jnp.bfloat16), b_ref[...],
                    preferred_element_type=jnp.float32)
    prod = e_all * gs_all
    q = prod[:, 0:NF]
    for s in range(1, N_DIST):
        q = q + prod[:, s * NF:(s + 1) * NF]

    rh = rh_ref[0] * cutsq
    c0_ref[...] = rh[:, 0:1] * q
    c1_ref[...] = rh[:, 1:2] * q
    c2_ref[...] = rh[:, 2:3] * q
    c3_ref[...] = rh[:, 3:4] * q


def _tc_main(g, dist3, rhat3, mu2, sg2, wcat, bsel):
    return pl.pallas_call(
        _tc_main_body,
        grid=(NB,),
        in_specs=[
            pl.BlockSpec((C, NF), lambda i: (i, 0)),
            pl.BlockSpec((1, 1, C), lambda i: (i, 0, 0)),
            pl.BlockSpec((1, C, 8), lambda i: (i, 0, 0)),
            pl.BlockSpec((1, N_DIST), lambda i: (0, 0)),
            pl.BlockSpec((1, N_DIST), lambda i: (0, 0)),
            pl.BlockSpec((NF, N_DIST * NF), lambda i: (0, 0)),
            pl.BlockSpec((N_DIST, N_DIST * NF), lambda i: (0, 0)),
        ],
        out_specs=[pl.BlockSpec((C, NF), lambda i: (i, 0))] * 4,
        out_shape=[jax.ShapeDtypeStruct((P_PAD, NF), jnp.float32)] * 4,
        compiler_params=pltpu.CompilerParams(
            dimension_semantics=("arbitrary",),
        ),
    )(g, dist3, rhat3, mu2, sg2, wcat, bsel)


def _sc_scatter(c0, c1, c2, c3, pf, zrows):
    mesh = plsc.VectorSubcoreMesh(core_axis_name="c", subcore_axis_name="s")

    @functools.partial(
        pl.kernel,
        mesh=mesh,
        out_type=jax.ShapeDtypeStruct((2, 4, PADROWS, NF), jnp.float32),
        scratch_types=[
            pltpu.VMEM((SCH, NF), jnp.float32),
            pltpu.VMEM((SCH, NF), jnp.float32),
            pltpu.VMEM((PFB,), jnp.int32),
            pltpu.VMEM((N_SCH, 1, SCH), jnp.int32),
            pltpu.VMEM_SHARED((ROWS, NF), jnp.float32),
            pltpu.SemaphoreType.DMA,
            pltpu.SemaphoreType.DMA,
        ],
    )
    def scatter_kernel(c0_hbm, c1_hbm, c2_hbm, c3_hbm, pf_hbm, z_hbm, out_hbm,
                       buf_a, buf_b, pfb, idx2, acc, sem_a, sem_b):
        ph = lax.axis_index("c")
        sid = lax.axis_index("s")
        base = ph * PPH + sid * PPT
        c_refs = (c0_hbm, c1_hbm, c2_hbm, c3_hbm)

        pltpu.sync_copy(pf_hbm.at[pl.ds(base, PFB)], pfb)

        def prep(j, carry):
            for kk in range(SCH // 16):
                pos = j * SCH + kk * 16 + lax.iota(jnp.int32, 16)
                v = pfb[pl.ds(j * SCH + kk * 16, 16)]
                inr = (v >= 0) & (v < N_ATOMS) & (pos < PPT)
                dummy = PADROWS + (v & (DUMROWS - 1))
                idx2[j, 0, pl.ds(kk * 16, 16)] = jnp.where(inr, v, dummy)
            return carry

        lax.fori_loop(0, N_SCH, prep, 0)

        for tp in range(4):
            pltpu.sync_copy(z_hbm, acc.at[pl.ds(sid * ZR, ZR)])
            plsc.subcore_barrier()
            src = c_refs[tp]

            pltpu.async_copy(src.at[pl.ds(base, SCH), :], buf_a, sem_a)

            def two(jj, carry):
                j1 = 2 * jj + 1
                pltpu.async_copy(src.at[pl.ds(base + j1 * SCH, SCH), :],
                                 buf_b, sem_b)
                pltpu.make_async_copy(src.at[pl.ds(base, SCH), :],
                                      buf_a, sem_a).wait()
                pltpu.sync_copy(buf_a, acc.at[idx2.at[j1 - 1, 0]], add=True)
                j2 = 2 * jj + 2
                pltpu.async_copy(src.at[pl.ds(base + j2 * SCH, SCH), :],
                                 buf_a, sem_a)
                pltpu.make_async_copy(src.at[pl.ds(base, SCH), :],
                                      buf_b, sem_b).wait()
                pltpu.sync_copy(buf_b, acc.at[idx2.at[j1, 0]], add=True)
                return carry

            lax.fori_loop(0, (N_SCH - 1) // 2, two, 0)
            pltpu.make_async_copy(src.at[pl.ds(base, SCH), :],
                                  buf_a, sem_a).wait()
            pltpu.sync_copy(buf_a, acc.at[idx2.at[N_SCH - 1, 0]], add=True)

            plsc.subcore_barrier()
            pltpu.sync_copy(acc.at[pl.ds(sid * OR_, OR_)],
                            out_hbm.at[ph, tp, pl.ds(sid * OR_, OR_)])
            plsc.subcore_barrier()

    return scatter_kernel(c0, c1, c2, c3, pf, zrows)


AB = PADROWS // 4


def _tc_tail_body(tf_ref, feat_ref, sw_ref, sb_ref, mw_ref, gg_ref, gb_ref, o_ref):
    tfr = tf_ref[0] + tf_ref[1]
    t0 = tfr[0]
    t1 = tfr[1]
    t2 = tfr[2]
    t3 = tfr[3]
    inv1 = t0
    inv2 = t1 * t1 + t2 * t2 + t3 * t3
    acc = jnp.zeros((AB, NF), jnp.float32)
    for gidx, xg in ((0, inv1), (1, inv2)):
        m = jnp.mean(xg, axis=1, keepdims=True)
        xc = xg - m
        v = jnp.mean(xc * xc, axis=1, keepdims=True)
        xn = xc * lax.rsqrt(v + GN_EPS)
        xn = xn * gg_ref[gidx:gidx + 1, :] + gb_ref[gidx:gidx + 1, :]
        mg = mw_ref[:, gidx, :]
        acc = acc + jnp.dot(xn, mg, preferred_element_type=jnp.float32)
    selfp = lax.dot_general(feat_ref[...], sw_ref[...], (((1,), (1,)), ((), ())),
                            preferred_element_type=jnp.float32) + sb_ref[...]
    o_ref[...] = acc + selfp


def _tc_tail(tfq, feat, sw, sb2, mw, gg2, gb2):
    nblk = 4
    return pl.pallas_call(
        _tc_tail_body,
        grid=(nblk,),
        in_specs=[
            pl.BlockSpec((2, 4, AB, NF), lambda i: (0, 0, i, 0)),
            pl.BlockSpec((AB, NF), lambda i: (i, 0)),
            pl.BlockSpec((NF, NF), lambda i: (0, 0)),
            pl.BlockSpec((1, NF), lambda i: (0, 0)),
            pl.BlockSpec((NF, 2, NF), lambda i: (0, 0, 0)),
            pl.BlockSpec((2, NF), lambda i: (0, 0)),
            pl.BlockSpec((2, NF), lambda i: (0, 0)),
        ],
        out_specs=pl.BlockSpec((AB, NF), lambda i: (i, 0)),
        out_shape=jax.ShapeDtypeStruct((N_ATOMS, NF), jnp.float32),
        compiler_params=pltpu.CompilerParams(
            dimension_semantics=("arbitrary",),
        ),
    )(tfq, feat, sw, sb2, mw, gg2, gb2)


def kernel(in_features, pair_first, pair_second, dist_pairs, tensor_rhats,
           sense_mu, sense_sigma, int_weights, selfint_W, selfint_b,
           mixing_weights, gn_gamma, gn_beta):
    g = _sc_gather(in_features, pair_second)

    dist3 = dist_pairs.reshape(NB, 1, C)
    rhat8 = jnp.concatenate(
        [tensor_rhats, jnp.zeros((N_PAIRS, 4), jnp.float32)], axis=1)
    rhat3 = rhat8.reshape(NB, C, 8)
    mu2 = sense_mu.reshape(1, N_DIST)
    sg2 = sense_sigma.reshape(1, N_DIST)

    wcat = int_weights.transpose(2, 0, 1).reshape(NF, N_DIST * NF).astype(jnp.bfloat16)
    bsel = (jnp.arange(N_DIST * NF)[None, :] // NF ==
            jnp.arange(N_DIST)[:, None]).astype(jnp.bfloat16)
    c0, c1, c2, c3 = _tc_main(g, dist3, rhat3, mu2, sg2, wcat, bsel)

    zrows = jnp.zeros((ZR, NF), jnp.float32)
    pf_pad = jnp.concatenate(
        [pair_first, jnp.full((P_PAD - N_PAIRS,), 1 << 29, jnp.int32)])
    tfq = _sc_scatter(c0, c1, c2, c3, pf_pad, zrows)

    out = _tc_tail(tfq, in_features, selfint_W, selfint_b.reshape(1, NF),
                   mixing_weights, gn_gamma.reshape(2, NF),
                   gn_beta.reshape(2, NF))
    return out
